# baseline (speedup 1.0000x reference)
import functools

import jax
import jax.numpy as jnp
from jax import lax
from jax.experimental import pallas as pl
from jax.experimental.pallas import tpu as pltpu
from jax.experimental.pallas import tpu_sc as plsc

N = 10000
E = 320000
D_FEAT = 128
D_EDGE = 16
NTYPE_DIM = 64
ETYPE_DIM = 16
NODE_W = NTYPE_DIM + D_FEAT
EDGE_W = ETYPE_DIM + D_EDGE
NUM_NTYPES = 119
NUM_ETYPES = 22

NC = 2
NS = 16
NW = NC * NS
L = 16

EC = 2560
E_CHUNKS = E // EC
EU = E_CHUNKS // NW
E_TAILW = E_CHUNKS - EU * NW
EGROUPS = EC // L

FCH = 6400
F_CHUNKS = E // FCH

NCH = 128
N_CHUNKS = -(-N // NCH)
N_PAD = N_CHUNKS * NCH


def _sc_body(etypes, etab, embT,
             etab_v, eidx0, eidx1, est0, est1,
             si0, si1, so0, so1):
    wid = lax.axis_index("s") * NC + lax.axis_index("c")

    pltpu.sync_copy(etab, etab_v)

    eidx = (eidx0, eidx1)
    est = (est0, est1)
    s_idx = (si0, si1)
    s_out = (so0, so1)

    def e_issue_in(k, b):
        base = pl.multiple_of((wid + k * NW) * EC, 128)
        pltpu.async_copy(etypes.at[pl.ds(base, EC)], eidx[b], s_idx[b])

    def e_wait_idx(b):
        pltpu.make_async_copy(etypes.at[pl.ds(0, EC)], eidx[b], s_idx[b]).wait()

    def e_wait_out(b):
        pltpu.make_async_copy(est[b], embT.at[pl.ds(0, ETYPE_DIM), pl.ds(0, EC)], s_out[b]).wait()

    def e_vector(b):
        @plsc.parallel_loop(0, EGROUPS, unroll=2)
        def _group(g):
            ev = eidx[b][pl.ds(g * L, L)]
            for d in range(ETYPE_DIM):
                dv = jnp.full((L,), d, jnp.int32)
                vals = plsc.load_gather(etab_v, [ev, dv])
                est[b][d, pl.ds(g * L, L)] = vals

    def e_issue_out(k, b):
        base = pl.multiple_of((wid + k * NW) * EC, 128)
        pltpu.async_copy(est[b], embT.at[pl.ds(0, ETYPE_DIM), pl.ds(base, EC)], s_out[b])

    def e_pair(j, carry):
        k0 = j * 2

        @pl.when(k0 > 0)
        def _():
            e_wait_out(1)
        e_issue_in(k0 + 1, 1)
        e_wait_idx(0)
        e_vector(0)
        e_issue_out(k0, 0)

        e_wait_out(0)
        e_issue_in(k0 + 2, 0)
        e_wait_idx(1)
        e_vector(1)
        e_issue_out(k0 + 1, 1)
        return carry

    e_issue_in(0, 0)
    lax.fori_loop(0, (EU - 1) // 2, e_pair, 0)

    e_wait_out(1)

    @pl.when(wid < E_TAILW)
    def _():
        e_issue_in(EU, 1)
    e_wait_idx(0)
    e_vector(0)
    e_issue_out(EU - 1, 0)

    @pl.when(wid < E_TAILW)
    def _():
        e_wait_out(0)
        e_wait_idx(1)
        e_vector(1)
        e_issue_out(EU, 1)
        e_wait_out(1)

    @pl.when(wid >= E_TAILW)
    def _():
        e_wait_out(0)


def _tc_node_body(ntypes3_ref, x_ref, ntab_ref, out_ref):
    t = ntypes3_ref[0, 0, :]
    r_iota = lax.broadcasted_iota(jnp.int32, (NCH, NCH), 0)
    oh = (r_iota == t[None, :]).astype(jnp.float32)
    embT = lax.dot_general(ntab_ref[...], oh, (((0,), (0,)), ((), ())),
                           preferred_element_type=jnp.float32,
                           precision=lax.Precision.HIGHEST)
    out_ref[0:NTYPE_DIM, :] = embT
    out_ref[NTYPE_DIM:NODE_W, :] = x_ref[...].T


def _tc_feat_body(acc_ref, eattrT_ref, out_ref):
    del acc_ref
    out_ref[...] = eattrT_ref[...]


@functools.partial(jax.jit, static_argnames=())
def kernel(x, eattr, ntypes, etypes, ntype_table, etype_table):
    eattrT = jnp.transpose(eattr)

    run_sc = pl.kernel(
        _sc_body,
        out_type=jax.ShapeDtypeStruct((EDGE_W, E), jnp.float32),
        mesh=plsc.VectorSubcoreMesh(core_axis_name="c", subcore_axis_name="s"),
        compiler_params=pltpu.CompilerParams(use_tc_tiling_on_sc=True,
                                             needs_layout_passes=False),
        scratch_types=[
            pltpu.VMEM((NUM_ETYPES, ETYPE_DIM), jnp.float32),
            pltpu.VMEM((EC,), jnp.int32),
            pltpu.VMEM((EC,), jnp.int32),
            pltpu.VMEM((ETYPE_DIM, EC), jnp.float32),
            pltpu.VMEM((ETYPE_DIM, EC), jnp.float32),
            pltpu.SemaphoreType.DMA,
            pltpu.SemaphoreType.DMA,
            pltpu.SemaphoreType.DMA,
            pltpu.SemaphoreType.DMA,
        ],
    )
    ecatT_emb = run_sc(etypes.astype(jnp.int32), etype_table)

    ntypes3 = jnp.pad(ntypes.astype(jnp.int32), (0, N_PAD - N)).reshape(
        N_CHUNKS, 1, NCH)
    ntab_pad = jnp.pad(ntype_table, ((0, NCH - NUM_NTYPES), (0, 0)))
    xcatT = pl.pallas_call(
        _tc_node_body,
        grid=(N_CHUNKS,),
        in_specs=[
            pl.BlockSpec((1, 1, NCH), lambda c: (c, 0, 0)),
            pl.BlockSpec((NCH, D_FEAT), lambda c: (c, 0)),
            pl.BlockSpec((NCH, NTYPE_DIM), lambda c: (0, 0)),
        ],
        out_specs=pl.BlockSpec((NODE_W, NCH), lambda c: (0, c)),
        out_shape=jax.ShapeDtypeStruct((NODE_W, N_PAD), jnp.float32),
    )(ntypes3, x, ntab_pad)

    ecatT = pl.pallas_call(
        _tc_feat_body,
        grid=(F_CHUNKS,),
        in_specs=[
            pl.BlockSpec(memory_space=pl.ANY),
            pl.BlockSpec((D_EDGE, FCH), lambda c: (0, c)),
        ],
        out_specs=pl.BlockSpec((D_EDGE, FCH), lambda c: (1, c)),
        out_shape=jax.ShapeDtypeStruct((EDGE_W, E), jnp.float32),
        input_output_aliases={0: 0},
    )(ecatT_emb, eattrT)

    return (jnp.transpose(xcatT)[:N], jnp.transpose(ecatT))


# speedup vs baseline: 1.2342x; 1.2342x over previous
import functools

import jax
import jax.numpy as jnp
from jax import lax
from jax.experimental import pallas as pl
from jax.experimental.pallas import tpu as pltpu
from jax.experimental.pallas import tpu_sc as plsc

N = 10000
E = 320000
D_FEAT = 128
D_EDGE = 16
NTYPE_DIM = 64
ETYPE_DIM = 16
NODE_W = NTYPE_DIM + D_FEAT
EDGE_W = ETYPE_DIM + D_EDGE
NUM_NTYPES = 119
NUM_ETYPES = 22

NC = 2
NS = 16
NW = NC * NS
STRIDE17 = 17
L = 16

EC = 2560
E_CHUNKS = E // EC
EU = E_CHUNKS // NW
E_TAILW = E_CHUNKS - EU * NW
EGROUPS = EC // L

FCH = 6400
F_CHUNKS = E // FCH

NCH = 128
N_CHUNKS = -(-N // NCH)
N_PAD = N_CHUNKS * NCH


def _sc_body(etypes, etab, embT,
             etab_v, etab17_v, eidx0, eidx1, est0, est1,
             si0, si1, so0, so1):
    wid = lax.axis_index("s") * NC + lax.axis_index("c")
    iota = lax.broadcasted_iota(jnp.int32, (L,), 0)

    pltpu.sync_copy(etab, etab_v)
    for r in range(NUM_ETYPES):
        plsc.store_scatter(etab17_v, [iota + r * STRIDE17], etab_v[r, :])

    eidx = (eidx0, eidx1)
    est = (est0, est1)
    s_idx = (si0, si1)
    s_out = (so0, so1)

    def e_issue_in(k, b):
        base = pl.multiple_of((wid + k * NW) * EC, 128)
        pltpu.async_copy(etypes.at[pl.ds(base, EC)], eidx[b], s_idx[b])

    def e_wait_idx(b):
        pltpu.make_async_copy(etypes.at[pl.ds(0, EC)], eidx[b], s_idx[b]).wait()

    def e_wait_out(b):
        pltpu.make_async_copy(est[b], embT.at[pl.ds(0, ETYPE_DIM), pl.ds(0, EC)], s_out[b]).wait()

    def e_vector(b):
        @plsc.parallel_loop(0, EGROUPS, unroll=2)
        def _group(g):
            ev17 = eidx[b][pl.ds(g * L, L)] * STRIDE17
            for d in range(ETYPE_DIM):
                dv = jnp.full((L,), d, jnp.int32)
                vals = plsc.load_gather(etab17_v, [ev17 + dv])
                est[b][d, pl.ds(g * L, L)] = vals

    def e_issue_out(k, b):
        base = pl.multiple_of((wid + k * NW) * EC, 128)
        pltpu.async_copy(est[b], embT.at[pl.ds(0, ETYPE_DIM), pl.ds(base, EC)], s_out[b])

    def e_pair(j, carry):
        k0 = j * 2

        @pl.when(k0 > 0)
        def _():
            e_wait_out(1)
        e_issue_in(k0 + 1, 1)
        e_wait_idx(0)
        e_vector(0)
        e_issue_out(k0, 0)

        e_wait_out(0)
        e_issue_in(k0 + 2, 0)
        e_wait_idx(1)
        e_vector(1)
        e_issue_out(k0 + 1, 1)
        return carry

    e_issue_in(0, 0)
    lax.fori_loop(0, (EU - 1) // 2, e_pair, 0)

    e_wait_out(1)

    @pl.when(wid < E_TAILW)
    def _():
        e_issue_in(EU, 1)
    e_wait_idx(0)
    e_vector(0)
    e_issue_out(EU - 1, 0)

    @pl.when(wid < E_TAILW)
    def _():
        e_wait_out(0)
        e_wait_idx(1)
        e_vector(1)
        e_issue_out(EU, 1)
        e_wait_out(1)

    @pl.when(wid >= E_TAILW)
    def _():
        e_wait_out(0)


def _tc_node_body(ntypes3_ref, x_ref, ntab_ref, out_ref):
    t = ntypes3_ref[0, 0, :]
    r_iota = lax.broadcasted_iota(jnp.int32, (NCH, NCH), 0)
    oh = (r_iota == t[None, :]).astype(jnp.float32)
    embT = lax.dot_general(ntab_ref[...], oh, (((0,), (0,)), ((), ())),
                           preferred_element_type=jnp.float32,
                           precision=lax.Precision.HIGHEST)
    out_ref[0:NTYPE_DIM, :] = embT
    out_ref[NTYPE_DIM:NODE_W, :] = x_ref[...].T


def _tc_feat_body(acc_ref, eattrT_ref, out_ref):
    del acc_ref
    out_ref[...] = eattrT_ref[...]


@functools.partial(jax.jit, static_argnames=())
def kernel(x, eattr, ntypes, etypes, ntype_table, etype_table):
    eattrT = jnp.transpose(eattr)

    run_sc = pl.kernel(
        _sc_body,
        out_type=jax.ShapeDtypeStruct((EDGE_W, E), jnp.float32),
        mesh=plsc.VectorSubcoreMesh(core_axis_name="c", subcore_axis_name="s"),
        compiler_params=pltpu.CompilerParams(use_tc_tiling_on_sc=True,
                                             needs_layout_passes=False),
        scratch_types=[
            pltpu.VMEM((NUM_ETYPES, ETYPE_DIM), jnp.float32),
            pltpu.VMEM((NUM_ETYPES * STRIDE17,), jnp.float32),
            pltpu.VMEM((EC,), jnp.int32),
            pltpu.VMEM((EC,), jnp.int32),
            pltpu.VMEM((ETYPE_DIM, EC), jnp.float32),
            pltpu.VMEM((ETYPE_DIM, EC), jnp.float32),
            pltpu.SemaphoreType.DMA,
            pltpu.SemaphoreType.DMA,
            pltpu.SemaphoreType.DMA,
            pltpu.SemaphoreType.DMA,
        ],
    )
    ecatT_emb = run_sc(etypes.astype(jnp.int32), etype_table)

    ntypes3 = jnp.pad(ntypes.astype(jnp.int32), (0, N_PAD - N)).reshape(
        N_CHUNKS, 1, NCH)
    ntab_pad = jnp.pad(ntype_table, ((0, NCH - NUM_NTYPES), (0, 0)))
    xcatT = pl.pallas_call(
        _tc_node_body,
        grid=(N_CHUNKS,),
        in_specs=[
            pl.BlockSpec((1, 1, NCH), lambda c: (c, 0, 0)),
            pl.BlockSpec((NCH, D_FEAT), lambda c: (c, 0)),
            pl.BlockSpec((NCH, NTYPE_DIM), lambda c: (0, 0)),
        ],
        out_specs=pl.BlockSpec((NODE_W, NCH), lambda c: (0, c)),
        out_shape=jax.ShapeDtypeStruct((NODE_W, N_PAD), jnp.float32),
    )(ntypes3, x, ntab_pad)

    ecatT = pl.pallas_call(
        _tc_feat_body,
        grid=(F_CHUNKS,),
        in_specs=[
            pl.BlockSpec(memory_space=pl.ANY),
            pl.BlockSpec((D_EDGE, FCH), lambda c: (0, c)),
        ],
        out_specs=pl.BlockSpec((D_EDGE, FCH), lambda c: (1, c)),
        out_shape=jax.ShapeDtypeStruct((EDGE_W, E), jnp.float32),
        input_output_aliases={0: 0},
    )(ecatT_emb, eattrT)

    return (jnp.transpose(xcatT)[:N], jnp.transpose(ecatT))


# speedup vs baseline: 1.6882x; 1.3679x over previous
import functools

import jax
import jax.numpy as jnp
from jax import lax
from jax.experimental import pallas as pl
from jax.experimental.pallas import tpu as pltpu
from jax.experimental.pallas import tpu_sc as plsc

N = 10000
E = 320000
D_FEAT = 128
D_EDGE = 16
NTYPE_DIM = 64
ETYPE_DIM = 16
NODE_W = NTYPE_DIM + D_FEAT
EDGE_W = ETYPE_DIM + D_EDGE
NUM_NTYPES = 119
NUM_ETYPES = 22

NC = 2
NS = 16
NW = NC * NS
STRIDE17 = 17
L = 16

EC = 1280
E_CHUNKS = E // EC
EU = E_CHUNKS // NW
E_TAILW = E_CHUNKS - EU * NW
EGROUPS = EC // L

NCH = 128
N_CHUNKS = -(-N // NCH)
N_PAD = N_CHUNKS * NCH


def _sc_body(eattrT, etypes, etab, ecatT,
             etab_v, etab17_v, eidx0, eidx1, est0, est1,
             si0, si1, sf0, sf1, so0, so1):
    wid = lax.axis_index("s") * NC + lax.axis_index("c")
    iota = lax.broadcasted_iota(jnp.int32, (L,), 0)

    pltpu.sync_copy(etab, etab_v)
    for r in range(NUM_ETYPES):
        plsc.store_scatter(etab17_v, [iota + r * STRIDE17], etab_v[r, :])

    eidx = (eidx0, eidx1)
    est = (est0, est1)
    s_idx = (si0, si1)
    s_feat = (sf0, sf1)
    s_out = (so0, so1)

    def e_issue_in(k, b):
        base = pl.multiple_of((wid + k * NW) * EC, 128)
        pltpu.async_copy(etypes.at[pl.ds(base, EC)], eidx[b], s_idx[b])
        pltpu.async_copy(eattrT.at[:, pl.ds(base, EC)],
                         est[b].at[pl.ds(ETYPE_DIM, D_EDGE), :], s_feat[b])

    def e_wait_idx(b):
        pltpu.make_async_copy(etypes.at[pl.ds(0, EC)], eidx[b], s_idx[b]).wait()

    def e_wait_feat(b):
        pltpu.make_async_copy(eattrT.at[:, pl.ds(0, EC)],
                              est[b].at[pl.ds(ETYPE_DIM, D_EDGE), :],
                              s_feat[b]).wait()

    def e_wait_out(b):
        pltpu.make_async_copy(est[b], ecatT.at[:, pl.ds(0, EC)], s_out[b]).wait()

    def e_vector(b):
        @plsc.parallel_loop(0, EGROUPS, unroll=2)
        def _group(g):
            ev17 = eidx[b][pl.ds(g * L, L)] * STRIDE17
            for d in range(ETYPE_DIM):
                dv = jnp.full((L,), d, jnp.int32)
                vals = plsc.load_gather(etab17_v, [ev17 + dv])
                est[b][d, pl.ds(g * L, L)] = vals

    def e_issue_out(k, b):
        base = pl.multiple_of((wid + k * NW) * EC, 128)
        pltpu.async_copy(est[b], ecatT.at[:, pl.ds(base, EC)], s_out[b])

    def e_pair(j, carry):
        k0 = j * 2

        @pl.when(k0 > 0)
        def _():
            e_wait_out(1)
        e_issue_in(k0 + 1, 1)
        e_wait_idx(0)
        e_vector(0)
        e_wait_feat(0)
        e_issue_out(k0, 0)

        e_wait_out(0)
        e_issue_in(k0 + 2, 0)
        e_wait_idx(1)
        e_vector(1)
        e_wait_feat(1)
        e_issue_out(k0 + 1, 1)
        return carry

    e_issue_in(0, 0)
    lax.fori_loop(0, (EU - 1) // 2, e_pair, 0)

    e_wait_out(1)

    @pl.when(wid < E_TAILW)
    def _():
        e_issue_in(EU, 1)
    e_wait_idx(0)
    e_vector(0)
    e_wait_feat(0)
    e_issue_out(EU - 1, 0)

    @pl.when(wid < E_TAILW)
    def _():
        e_wait_out(0)
        e_wait_idx(1)
        e_vector(1)
        e_wait_feat(1)
        e_issue_out(EU, 1)
        e_wait_out(1)

    @pl.when(wid >= E_TAILW)
    def _():
        e_wait_out(0)


def _tc_node_body(ntypes3_ref, x_ref, ntab_ref, out_ref):
    t = ntypes3_ref[0, 0, :]
    r_iota = lax.broadcasted_iota(jnp.int32, (NCH, NCH), 0)
    oh = (r_iota == t[None, :]).astype(jnp.float32)
    embT = lax.dot_general(ntab_ref[...], oh, (((0,), (0,)), ((), ())),
                           preferred_element_type=jnp.float32,
                           precision=lax.Precision.HIGHEST)
    out_ref[0:NTYPE_DIM, :] = embT
    out_ref[NTYPE_DIM:NODE_W, :] = x_ref[...].T


@functools.partial(jax.jit, static_argnames=())
def kernel(x, eattr, ntypes, etypes, ntype_table, etype_table):
    eattrT = jnp.transpose(eattr)

    run_sc = pl.kernel(
        _sc_body,
        out_type=jax.ShapeDtypeStruct((EDGE_W, E), jnp.float32),
        mesh=plsc.VectorSubcoreMesh(core_axis_name="c", subcore_axis_name="s"),
        compiler_params=pltpu.CompilerParams(use_tc_tiling_on_sc=True,
                                             needs_layout_passes=False),
        scratch_types=[
            pltpu.VMEM((NUM_ETYPES, ETYPE_DIM), jnp.float32),
            pltpu.VMEM((NUM_ETYPES * STRIDE17,), jnp.float32),
            pltpu.VMEM((EC,), jnp.int32),
            pltpu.VMEM((EC,), jnp.int32),
            pltpu.VMEM((EDGE_W, EC), jnp.float32),
            pltpu.VMEM((EDGE_W, EC), jnp.float32),
            pltpu.SemaphoreType.DMA,
            pltpu.SemaphoreType.DMA,
            pltpu.SemaphoreType.DMA,
            pltpu.SemaphoreType.DMA,
            pltpu.SemaphoreType.DMA,
            pltpu.SemaphoreType.DMA,
        ],
    )
    ecatT = run_sc(eattrT, etypes.astype(jnp.int32), etype_table)

    ntypes3 = jnp.pad(ntypes.astype(jnp.int32), (0, N_PAD - N)).reshape(
        N_CHUNKS, 1, NCH)
    ntab_pad = jnp.pad(ntype_table, ((0, NCH - NUM_NTYPES), (0, 0)))
    xcatT = pl.pallas_call(
        _tc_node_body,
        grid=(N_CHUNKS,),
        in_specs=[
            pl.BlockSpec((1, 1, NCH), lambda c: (c, 0, 0)),
            pl.BlockSpec((NCH, D_FEAT), lambda c: (c, 0)),
            pl.BlockSpec((NCH, NTYPE_DIM), lambda c: (0, 0)),
        ],
        out_specs=pl.BlockSpec((NODE_W, NCH), lambda c: (0, c)),
        out_shape=jax.ShapeDtypeStruct((NODE_W, N_PAD), jnp.float32),
    )(ntypes3, x, ntab_pad)

    return (jnp.transpose(xcatT)[:N], jnp.transpose(ecatT))


# speedup vs baseline: 2.5740x; 1.5247x over previous
import functools

import jax
import jax.numpy as jnp
from jax import lax
from jax.experimental import pallas as pl
from jax.experimental.pallas import tpu as pltpu
from jax.experimental.pallas import tpu_sc as plsc

N = 10000
E = 320000
D_FEAT = 128
D_EDGE = 16
NTYPE_DIM = 64
ETYPE_DIM = 16
NODE_W = NTYPE_DIM + D_FEAT
EDGE_W = ETYPE_DIM + D_EDGE
NUM_NTYPES = 119
NUM_ETYPES = 22

NC = 2
NS = 16
NW = NC * NS
STRIDE17 = 17
L = 16

EC = 1280
E_CHUNKS = E // EC
EU = E_CHUNKS // NW
E_TAILW = E_CHUNKS - EU * NW
EGROUPS = EC // L

NCH = 128
N_CHUNKS = -(-N // NCH)
N_PAD = N_CHUNKS * NCH
CB = 1024
CB_CHUNKS = -(-N_PAD // CB)
N_PAD2 = CB_CHUNKS * CB


def _sc_body(eattrT, etypes, etab, ecatT,
             etab_v, etab17_v, eidx0, eidx1, est0, est1,
             si0, si1, sf0, sf1, so0, so1):
    wid = lax.axis_index("s") * NC + lax.axis_index("c")
    iota = lax.broadcasted_iota(jnp.int32, (L,), 0)

    pltpu.sync_copy(etab, etab_v)
    for r in range(NUM_ETYPES):
        plsc.store_scatter(etab17_v, [iota + r * STRIDE17], etab_v[r, :])

    eidx = (eidx0, eidx1)
    est = (est0, est1)
    s_idx = (si0, si1)
    s_feat = (sf0, sf1)
    s_out = (so0, so1)

    def e_issue_in(k, b):
        base = pl.multiple_of((wid + k * NW) * EC, 128)
        pltpu.async_copy(etypes.at[pl.ds(base, EC)], eidx[b], s_idx[b])
        pltpu.async_copy(eattrT.at[:, pl.ds(base, EC)],
                         est[b].at[pl.ds(ETYPE_DIM, D_EDGE), :], s_feat[b])

    def e_wait_idx(b):
        pltpu.make_async_copy(etypes.at[pl.ds(0, EC)], eidx[b], s_idx[b]).wait()

    def e_wait_feat(b):
        pltpu.make_async_copy(eattrT.at[:, pl.ds(0, EC)],
                              est[b].at[pl.ds(ETYPE_DIM, D_EDGE), :],
                              s_feat[b]).wait()

    def e_wait_out(b):
        pltpu.make_async_copy(est[b], ecatT.at[:, pl.ds(0, EC)], s_out[b]).wait()

    def e_vector(b):
        @plsc.parallel_loop(0, EGROUPS, unroll=2)
        def _group(g):
            ev17 = eidx[b][pl.ds(g * L, L)] * STRIDE17
            for d in range(ETYPE_DIM):
                dv = jnp.full((L,), d, jnp.int32)
                vals = plsc.load_gather(etab17_v, [ev17 + dv])
                est[b][d, pl.ds(g * L, L)] = vals

    def e_issue_out(k, b):
        base = pl.multiple_of((wid + k * NW) * EC, 128)
        pltpu.async_copy(est[b], ecatT.at[:, pl.ds(base, EC)], s_out[b])

    def e_pair(j, carry):
        k0 = j * 2

        @pl.when(k0 > 0)
        def _():
            e_wait_out(1)
        e_issue_in(k0 + 1, 1)
        e_wait_idx(0)
        e_vector(0)
        e_wait_feat(0)
        e_issue_out(k0, 0)

        e_wait_out(0)
        e_issue_in(k0 + 2, 0)
        e_wait_idx(1)
        e_vector(1)
        e_wait_feat(1)
        e_issue_out(k0 + 1, 1)
        return carry

    e_issue_in(0, 0)
    lax.fori_loop(0, (EU - 1) // 2, e_pair, 0)

    e_wait_out(1)

    @pl.when(wid < E_TAILW)
    def _():
        e_issue_in(EU, 1)
    e_wait_idx(0)
    e_vector(0)
    e_wait_feat(0)
    e_issue_out(EU - 1, 0)

    @pl.when(wid < E_TAILW)
    def _():
        e_wait_out(0)
        e_wait_idx(1)
        e_vector(1)
        e_wait_feat(1)
        e_issue_out(EU, 1)
        e_wait_out(1)

    @pl.when(wid >= E_TAILW)
    def _():
        e_wait_out(0)


def _tc_node_body(ntypes3_ref, x_ref, ntab_ref, out_ref):
    t = ntypes3_ref[0, 0, :]
    r_iota = lax.broadcasted_iota(jnp.int32, (NCH, CB), 0)
    oh = (r_iota == t[None, :]).astype(jnp.float32)
    embT = lax.dot_general(ntab_ref[...], oh, (((0,), (0,)), ((), ())),
                           preferred_element_type=jnp.float32,
                           precision=lax.Precision.HIGHEST)
    out_ref[0:NTYPE_DIM, :] = embT
    out_ref[NTYPE_DIM:NODE_W, :] = x_ref[...].T


@functools.partial(jax.jit, static_argnames=())
def kernel(x, eattr, ntypes, etypes, ntype_table, etype_table):
    eattrT = jnp.transpose(eattr)

    run_sc = pl.kernel(
        _sc_body,
        out_type=jax.ShapeDtypeStruct((EDGE_W, E), jnp.float32),
        mesh=plsc.VectorSubcoreMesh(core_axis_name="c", subcore_axis_name="s"),
        compiler_params=pltpu.CompilerParams(use_tc_tiling_on_sc=True,
                                             needs_layout_passes=False),
        scratch_types=[
            pltpu.VMEM((NUM_ETYPES, ETYPE_DIM), jnp.float32),
            pltpu.VMEM((NUM_ETYPES * STRIDE17,), jnp.float32),
            pltpu.VMEM((EC,), jnp.int32),
            pltpu.VMEM((EC,), jnp.int32),
            pltpu.VMEM((EDGE_W, EC), jnp.float32),
            pltpu.VMEM((EDGE_W, EC), jnp.float32),
            pltpu.SemaphoreType.DMA,
            pltpu.SemaphoreType.DMA,
            pltpu.SemaphoreType.DMA,
            pltpu.SemaphoreType.DMA,
            pltpu.SemaphoreType.DMA,
            pltpu.SemaphoreType.DMA,
        ],
    )
    ecatT = run_sc(eattrT, etypes.astype(jnp.int32), etype_table)

    ntypes3 = jnp.pad(ntypes.astype(jnp.int32), (0, N_PAD2 - N)).reshape(
        CB_CHUNKS, 1, CB)
    ntab_pad = jnp.pad(ntype_table, ((0, NCH - NUM_NTYPES), (0, 0)))
    xcatT = pl.pallas_call(
        _tc_node_body,
        grid=(CB_CHUNKS,),
        in_specs=[
            pl.BlockSpec((1, 1, CB), lambda c: (c, 0, 0)),
            pl.BlockSpec((CB, D_FEAT), lambda c: (c, 0)),
            pl.BlockSpec((NCH, NTYPE_DIM), lambda c: (0, 0)),
        ],
        out_specs=pl.BlockSpec((NODE_W, CB), lambda c: (0, c)),
        out_shape=jax.ShapeDtypeStruct((NODE_W, N_PAD), jnp.float32),
    )(ntypes3, x, ntab_pad)

    return (jnp.transpose(xcatT)[:N], jnp.transpose(ecatT))
